# Initial kernel scaffold; baseline (speedup 1.0000x reference)
import jax
import jax.numpy as jnp
from jax.experimental import pallas as pl


def kernel(raw_preds, targets, epoch):
    raise NotImplementedError("write your pallas kernel here")


# speedup vs baseline: 1.2723x; 1.2723x over previous
import jax
import jax.numpy as jnp
from jax.experimental import pallas as pl
from jax.experimental.pallas import tpu as pltpu

_LAMBDA_BOX = 5.0
_CHUNK = 2048


def _dl_kernel(pbcT_ref, raw_ref, tgt_ref, out_ref, g_scr):
    P = raw_ref.shape[1]
    C = raw_ref.shape[2]
    T = tgt_ref.shape[1]

    tgt = tgt_ref[0]
    tx1 = tgt[:, 0:1]
    ty1 = tgt[:, 1:2]
    tx2 = tgt[:, 2:3]
    ty2 = tgt[:, 3:4]
    area_t = (tx2 - tx1) * (ty2 - ty1)

    run_max = jnp.full((T, 1), -jnp.inf, jnp.float32)
    run_idx = jnp.zeros((T, 1), jnp.int32)
    sp_acc = jnp.zeros((1, _CHUNK), jnp.float32)

    for i in range(P // _CHUNK):
        off = i * _CHUNK
        px1 = pbcT_ref[0, 0:1, off:off + _CHUNK]
        py1 = pbcT_ref[0, 1:2, off:off + _CHUNK]
        px2 = pbcT_ref[0, 2:3, off:off + _CHUNK]
        py2 = pbcT_ref[0, 3:4, off:off + _CHUNK]
        cf = pbcT_ref[0, 4:5, off:off + _CHUNK]
        w = jnp.minimum(px2, tx2) - jnp.maximum(px1, tx1)
        h = jnp.minimum(py2, ty2) - jnp.maximum(py1, ty1)
        inter = w * h
        area_p = (px2 - px1) * (py2 - py1)
        union = (area_p + area_t) - inter
        iou = jnp.where((w > 0.0) & (h > 0.0), inter / union, 0.0)
        lmax = jnp.max(iou, axis=1, keepdims=True)
        larg = jnp.argmax(iou, axis=1, keepdims=True).astype(jnp.int32)
        upd = lmax > run_max
        run_max = jnp.where(upd, lmax, run_max)
        run_idx = jnp.where(upd, larg + off, run_idx)
        sp_acc = sp_acc + jnp.logaddexp(0.0, cf)

    ri_row = jnp.swapaxes(run_idx, 0, 1)
    sub_iota = jax.lax.broadcasted_iota(jnp.int32, (8, C), 0)
    for t in range(T):
        idx = ri_row[0, t]
        base = pl.multiple_of((idx >> 3) << 3, 8)
        chunk = raw_ref[0, pl.ds(base, 8), :]
        sel = sub_iota == (idx & 7)
        g_scr[t:t + 1, :] = jnp.sum(jnp.where(sel, chunk, 0.0), axis=0,
                                    keepdims=True)

    g = g_scr[:, :]

    d = jnp.abs(g[:, 0:4] - tgt[:, 0:4])
    box_loss = jnp.sum(jnp.where(d < 1.0, 0.5 * d * d, d - 0.5),
                       axis=(0, 1), keepdims=True)

    logits = g[:, 5:]
    m = jnp.max(logits, axis=1, keepdims=True)
    lse = m + jnp.log(jnp.sum(jnp.exp(logits - m), axis=1, keepdims=True))
    tcls = tgt[:, 4:5].astype(jnp.int32)
    cls_iota = jax.lax.broadcasted_iota(jnp.int32, (T, C - 5), 1)
    logit_t = jnp.sum(jnp.where(cls_iota == tcls, logits, 0.0), axis=1,
                      keepdims=True)
    cls_loss = jnp.sum(lse - logit_t, axis=(0, 1), keepdims=True)

    x = g[:, 4:5]
    eq = run_idx == ri_row
    li = jax.lax.broadcasted_iota(jnp.int32, (T, T), 1)
    ti = jax.lax.broadcasted_iota(jnp.int32, (T, T), 0)
    dup = jnp.sum(jnp.where(eq & (li < ti), 1.0, 0.0), axis=1,
                  keepdims=True) > 0.0
    conf_sub = jnp.sum(jnp.where(dup, 0.0, x), axis=(0, 1), keepdims=True)
    sp_total = jnp.sum(sp_acc, axis=(0, 1), keepdims=True)

    out_ref[0] = (_LAMBDA_BOX * box_loss + cls_loss
                  + sp_total - conf_sub)


def kernel(raw_preds, targets, epoch):
    del epoch
    B, P, C = raw_preds.shape
    T = targets.shape[1]
    pbcT = jnp.swapaxes(raw_preds[..., :5], 1, 2)
    per_sample = pl.pallas_call(
        _dl_kernel,
        grid=(B,),
        in_specs=[
            pl.BlockSpec((1, 5, P), lambda b: (b, 0, 0)),
            pl.BlockSpec((1, P, C), lambda b: (b, 0, 0)),
            pl.BlockSpec((1, T, 5), lambda b: (b, 0, 0)),
        ],
        out_specs=pl.BlockSpec((1, 1, 1), lambda b: (b, 0, 0)),
        out_shape=jax.ShapeDtypeStruct((B, 1, 1), jnp.float32),
        scratch_shapes=[pltpu.VMEM((T, C), jnp.float32)],
        compiler_params=pltpu.CompilerParams(
            dimension_semantics=("parallel",),
        ),
    )(pbcT, raw_preds, targets)
    return jnp.sum(per_sample) / B
